# Initial kernel scaffold; baseline (speedup 1.0000x reference)
import jax
import jax.numpy as jnp
from jax.experimental import pallas as pl


def kernel(x, point_pairs, cluster_ids, recons, pts):
    raise NotImplementedError("write your pallas kernel here")


# speedup vs baseline: 1.0103x; 1.0103x over previous
import jax
import jax.numpy as jnp
from jax.experimental import pallas as pl
from jax.experimental.pallas import tpu as pltpu

_TAU = 0.07


def _loss_body(exp_ref, den_ref, pos_ref, out_ref):
    e = exp_ref[...]
    d = den_ref[...]
    p = pos_ref[...]
    lp = -jnp.log(e / (e + d))
    out_ref[0, 0] = jnp.sum(lp * p) / jnp.sum(p)


def kernel(x, point_pairs, cluster_ids, recons, pts):
    src = point_pairs[0]
    dst = point_pairs[1]
    ci0 = cluster_ids[src]
    ci1 = cluster_ids[dst]
    pos = (ci0 == ci1) & (ci0 != -1) & (ci1 != -1)
    a = jnp.take(x, src, axis=0)
    b = jnp.take(x, dst, axis=0)
    na = jnp.sqrt(jnp.sum(a * a, axis=-1))
    nb = jnp.sqrt(jnp.sum(b * b, axis=-1))
    sim = jnp.sum(a * b, axis=-1) / jnp.maximum(na * nb, 1e-8)
    ex = jnp.exp(sim / _TAU - 1.0 / _TAU)
    den_full = jax.ops.segment_sum(jnp.where(pos, 0.0, ex), src,
                                   num_segments=x.shape[0])
    den = jnp.maximum(den_full, 0.0)[src]

    E = src.shape[0]
    W = 512
    exp2 = ex.reshape(E // W, W)
    den2 = den.reshape(E // W, W)
    pos2 = pos.astype(jnp.float32).reshape(E // W, W)
    out = pl.pallas_call(
        _loss_body,
        out_shape=jax.ShapeDtypeStruct((1, 1), jnp.float32),
        out_specs=pl.BlockSpec(memory_space=pltpu.SMEM),
    )(exp2, den2, pos2)
    return out[0, 0]


# speedup vs baseline: 5.6572x; 5.5998x over previous
import functools

import jax
import jax.numpy as jnp
from jax import lax
from jax.experimental import pallas as pl
from jax.experimental.pallas import tpu as pltpu
from jax.experimental.pallas import tpu_sc as plsc

_TAU = 0.07
_INV_TAU = 1.0 / 0.07

_N = 10000
_E = 320000
_D = 128
_NC = 2
_NS = 16
_NW = _NC * _NS
_EPT = _E // _NW
_BA = 80
_GA = _BA // 16
_NCHUNK_A = _EPT // _BA
_BB = 2000
_NCHUNK_B = _EPT // _BB


def _norms_body(x_ref, na_ref):
    x = x_ref[...]
    na_ref[...] = jnp.sqrt(jnp.sum(x * x, axis=1, keepdims=True))


def _loss_body(exp_ref, den_ref, pos_ref, out_ref):
    e = exp_ref[...]
    d = den_ref[...]
    p = pos_ref[...]
    lp = -jnp.log(e / (e + d))
    out_ref[0, 0] = jnp.sum(lp * p) / jnp.sum(p)


def _ka_body(x_hbm, src_hbm, dst_hbm, ci_hbm, na_hbm,
             exp_out, pos_out, den2_out,
             na_tab, ci_tab, src_buf, dst_buf, rows_a, rows_b,
             exp_buf, pos_buf, neg_buf, den_sh, sem_a, sem_b):
    cid = lax.axis_index("c")
    sid = lax.axis_index("s")
    wid = sid * _NC + cid
    iota = lax.iota(jnp.int32, 16)

    @pl.when(sid == 0)
    def _():
        def zbody(i, _):
            na_tab[pl.ds(i * 16, 16)] = jnp.zeros((16,), jnp.float32)
            return 0
        lax.fori_loop(0, _N // 16, zbody, 0, unroll=8)
        pltpu.sync_copy(na_tab, den_sh)

    plsc.subcore_barrier()

    pltpu.sync_copy(na_hbm, na_tab)
    pltpu.sync_copy(ci_hbm, ci_tab)

    def chunk_body(k, _):
        off = pl.multiple_of(wid * _EPT + k * _BA, 8)
        pltpu.sync_copy(src_hbm.at[pl.ds(off, _BA)], src_buf)
        pltpu.sync_copy(dst_hbm.at[pl.ds(off, _BA)], dst_buf)
        cp_a = pltpu.async_copy(x_hbm.at[src_buf], rows_a, sem_a)
        cp_b = pltpu.async_copy(x_hbm.at[dst_buf], rows_b, sem_b)
        cp_a.wait()
        cp_b.wait()

        for g in range(_GA):
            ridx = iota + (g * 16)
            srcv = src_buf[pl.ds(g * 16, 16)]
            dstv = dst_buf[pl.ds(g * 16, 16)]

            def cbody(c, acc):
                col = jnp.full((16,), c, jnp.int32)
                av = plsc.load_gather(rows_a, [ridx, col])
                bv = plsc.load_gather(rows_b, [ridx, col])
                return acc + av * bv

            dot = lax.fori_loop(0, _D, cbody, jnp.zeros((16,), jnp.float32),
                                unroll=8)
            nas = plsc.load_gather(na_tab, [srcv])
            nad = plsc.load_gather(na_tab, [dstv])
            cis = plsc.load_gather(ci_tab, [srcv])
            cid_v = plsc.load_gather(ci_tab, [dstv])
            sim = dot / jnp.maximum(nas * nad, 1e-8)
            ex = jnp.exp(sim * _INV_TAU - _INV_TAU)
            posm = (cis == cid_v) & (cis != -1) & (cid_v != -1)
            exp_buf[pl.ds(g * 16, 16)] = ex
            pos_buf[pl.ds(g * 16, 16)] = jnp.where(posm, 1.0, 0.0)
            neg_buf[pl.ds(g * 16, 16)] = jnp.where(posm, 0.0, ex)

        pltpu.sync_copy(exp_buf, exp_out.at[pl.ds(off, _BA)])
        pltpu.sync_copy(pos_buf, pos_out.at[pl.ds(off, _BA)])
        pltpu.sync_copy(neg_buf, den_sh.at[src_buf], add=True)
        return 0

    lax.fori_loop(0, _NCHUNK_A, chunk_body, 0)

    plsc.subcore_barrier()

    @pl.when(sid == 0)
    def _():
        pltpu.sync_copy(den_sh, den2_out.at[cid])


def _kb_body(den2_hbm, src_hbm, den_out, dtab, t1, sbuf, obuf):
    cid = lax.axis_index("c")
    sid = lax.axis_index("s")
    wid = sid * _NC + cid
    pltpu.sync_copy(den2_hbm.at[0], dtab)
    pltpu.sync_copy(den2_hbm.at[1], t1)

    def addb(i, _):
        o = i * 16
        dtab[pl.ds(o, 16)] = jnp.maximum(
            dtab[pl.ds(o, 16)] + t1[pl.ds(o, 16)], 0.0)
        return 0

    lax.fori_loop(0, _N // 16, addb, 0, unroll=8)

    def chunk_body(k, _):
        off = pl.multiple_of(wid * _EPT + k * _BB, 8)
        pltpu.sync_copy(src_hbm.at[pl.ds(off, _BB)], sbuf)

        def gb(g, _):
            o = g * 16
            sv = sbuf[pl.ds(o, 16)]
            obuf[pl.ds(o, 16)] = plsc.load_gather(dtab, [sv])
            return 0

        lax.fori_loop(0, _BB // 16, gb, 0, unroll=8)
        pltpu.sync_copy(obuf, den_out.at[pl.ds(off, _BB)])
        return 0

    lax.fori_loop(0, _NCHUNK_B, chunk_body, 0)


_SC_MESH = plsc.VectorSubcoreMesh(core_axis_name="c", subcore_axis_name="s")

_ka = functools.partial(
    pl.kernel,
    mesh=_SC_MESH,
    compiler_params=pltpu.CompilerParams(needs_layout_passes=False),
    out_type=[
        jax.ShapeDtypeStruct((_E,), jnp.float32),
        jax.ShapeDtypeStruct((_E,), jnp.float32),
        jax.ShapeDtypeStruct((_NC, _N), jnp.float32),
    ],
    scratch_types=[
        pltpu.VMEM((_N,), jnp.float32),
        pltpu.VMEM((_N,), jnp.int32),
        pltpu.VMEM((_BA,), jnp.int32),
        pltpu.VMEM((_BA,), jnp.int32),
        pltpu.VMEM((_BA, _D), jnp.float32),
        pltpu.VMEM((_BA, _D), jnp.float32),
        pltpu.VMEM((_BA,), jnp.float32),
        pltpu.VMEM((_BA,), jnp.float32),
        pltpu.VMEM((_BA,), jnp.float32),
        pltpu.VMEM_SHARED((_N,), jnp.float32),
        pltpu.SemaphoreType.DMA,
        pltpu.SemaphoreType.DMA,
    ],
)(_ka_body)

_kb = functools.partial(
    pl.kernel,
    mesh=_SC_MESH,
    compiler_params=pltpu.CompilerParams(needs_layout_passes=False),
    out_type=jax.ShapeDtypeStruct((_E,), jnp.float32),
    scratch_types=[
        pltpu.VMEM((_N,), jnp.float32),
        pltpu.VMEM((_N,), jnp.float32),
        pltpu.VMEM((_BB,), jnp.int32),
        pltpu.VMEM((_BB,), jnp.float32),
    ],
)(_kb_body)


def kernel(x, point_pairs, cluster_ids, recons, pts):
    src = point_pairs[0]
    dst = point_pairs[1]
    na2d = pl.pallas_call(
        _norms_body,
        out_shape=jax.ShapeDtypeStruct((_N, 1), jnp.float32),
    )(x)
    na = na2d.reshape(_N)

    expv, posf, den2 = _ka(x, src, dst, cluster_ids.astype(jnp.int32), na)
    den = _kb(den2, src)

    W = 128
    out = pl.pallas_call(
        _loss_body,
        out_shape=jax.ShapeDtypeStruct((1, 1), jnp.float32),
        out_specs=pl.BlockSpec(memory_space=pltpu.SMEM),
    )(expv.reshape(_E // W, W), den.reshape(_E // W, W),
      posf.reshape(_E // W, W))
    return out[0, 0]


# speedup vs baseline: 6.8276x; 1.2069x over previous
import functools

import jax
import jax.numpy as jnp
from jax import lax
from jax.experimental import pallas as pl
from jax.experimental.pallas import tpu as pltpu
from jax.experimental.pallas import tpu_sc as plsc

_TAU = 0.07
_INV_TAU = 1.0 / 0.07

_N = 10000
_E = 320000
_D = 128
_NC = 2
_NS = 16
_NW = _NC * _NS
_EPT = _E // _NW
_BB = 2000
_NOUT = _EPT // _BB
_BS = 80
_GS = _BS // 16
_NSUB = _BB // _BS


def _norms_body(x_ref, na_ref):
    x = x_ref[...]
    na_ref[...] = jnp.sqrt(jnp.sum(x * x, axis=1, keepdims=True))


def _loss_body(exp_ref, den_ref, pos_ref, out_ref):
    e = exp_ref[...]
    d = den_ref[...]
    p = pos_ref[...]
    lp = -jnp.log(e / (e + d))
    out_ref[0, 0] = jnp.sum(lp * p) / jnp.sum(p)


def _ka_body(x_hbm, src_hbm, dst_hbm, ci_hbm, na_hbm,
             exp_out, pos_out, den2_out,
             na_tab, ci_tab, src_big, dst_big,
             rows_a0, rows_b0, rows_a1, rows_b1,
             exp_big, pos_big, neg_big, den_sh,
             sem_a0, sem_b0, sem_a1, sem_b1):
    cid = lax.axis_index("c")
    sid = lax.axis_index("s")
    wid = sid * _NC + cid
    iota = lax.iota(jnp.int32, 16)
    rows = ((rows_a0, rows_b0, sem_a0, sem_b0),
            (rows_a1, rows_b1, sem_a1, sem_b1))

    @pl.when(sid == 0)
    def _():
        def zbody(i, _):
            na_tab[pl.ds(i * 16, 16)] = jnp.zeros((16,), jnp.float32)
            return 0
        lax.fori_loop(0, _N // 16, zbody, 0, unroll=8)
        pltpu.sync_copy(na_tab, den_sh)

    plsc.subcore_barrier()

    pltpu.sync_copy(na_hbm, na_tab)
    pltpu.sync_copy(ci_hbm, ci_tab)

    def fire(j, p):
        ra, rb, sa, sb = rows[p]
        idx_s = src_big.at[pl.ds(j * _BS, _BS)]
        idx_d = dst_big.at[pl.ds(j * _BS, _BS)]
        pltpu.make_async_copy(x_hbm.at[idx_s], ra, sa).start()
        pltpu.make_async_copy(x_hbm.at[idx_d], rb, sb).start()

    def compute(j, p):
        ra, rb, sa, sb = rows[p]
        pltpu.make_async_copy(x_hbm.at[src_big.at[pl.ds(0, _BS)]], ra,
                              sa).wait()
        pltpu.make_async_copy(x_hbm.at[dst_big.at[pl.ds(0, _BS)]], rb,
                              sb).wait()
        for g in range(_GS):
            ridx = iota + (g * 16)
            eoff = j * _BS + g * 16
            srcv = src_big[pl.ds(eoff, 16)]
            dstv = dst_big[pl.ds(eoff, 16)]

            def cbody(c, acc):
                col = jnp.full((16,), c, jnp.int32)
                av = plsc.load_gather(ra, [ridx, col])
                bv = plsc.load_gather(rb, [ridx, col])
                return acc + av * bv

            dot = lax.fori_loop(0, _D, cbody, jnp.zeros((16,), jnp.float32),
                                unroll=8)
            nas = plsc.load_gather(na_tab, [srcv])
            nad = plsc.load_gather(na_tab, [dstv])
            cis = plsc.load_gather(ci_tab, [srcv])
            cid_v = plsc.load_gather(ci_tab, [dstv])
            sim = dot / jnp.maximum(nas * nad, 1e-8)
            ex = jnp.exp(sim * _INV_TAU - _INV_TAU)
            posm = (cis == cid_v) & (cis != -1) & (cid_v != -1)
            exp_big[pl.ds(eoff, 16)] = ex
            pos_big[pl.ds(eoff, 16)] = jnp.where(posm, 1.0, 0.0)
            neg_big[pl.ds(eoff, 16)] = jnp.where(posm, 0.0, ex)

    def outer_body(ko, _):
        off = pl.multiple_of(wid * _EPT + ko * _BB, 8)
        pltpu.sync_copy(src_hbm.at[pl.ds(off, _BB)], src_big)
        pltpu.sync_copy(dst_hbm.at[pl.ds(off, _BB)], dst_big)

        fire(0, 0)

        def sub_body(m, _):
            j = m * 2
            fire(j + 1, 1)
            compute(j, 0)

            @pl.when(j + 2 < _NSUB)
            def _():
                fire(j + 2, 0)

            compute(j + 1, 1)
            return 0

        lax.fori_loop(0, (_NSUB - 1) // 2, sub_body, 0)
        compute(_NSUB - 1, 0)

        pltpu.sync_copy(exp_big, exp_out.at[pl.ds(off, _BB)])
        pltpu.sync_copy(pos_big, pos_out.at[pl.ds(off, _BB)])
        pltpu.sync_copy(neg_big, den_sh.at[src_big], add=True)
        return 0

    lax.fori_loop(0, _NOUT, outer_body, 0)

    plsc.subcore_barrier()

    @pl.when(sid == 0)
    def _():
        pltpu.sync_copy(den_sh, den2_out.at[cid])


def _kb_body(den2_hbm, src_hbm, den_out, dtab, t1, sbuf, obuf):
    cid = lax.axis_index("c")
    sid = lax.axis_index("s")
    wid = sid * _NC + cid
    pltpu.sync_copy(den2_hbm.at[0], dtab)
    pltpu.sync_copy(den2_hbm.at[1], t1)

    def addb(i, _):
        o = i * 16
        dtab[pl.ds(o, 16)] = jnp.maximum(
            dtab[pl.ds(o, 16)] + t1[pl.ds(o, 16)], 0.0)
        return 0

    lax.fori_loop(0, _N // 16, addb, 0, unroll=8)

    def chunk_body(k, _):
        off = pl.multiple_of(wid * _EPT + k * _BB, 8)
        pltpu.sync_copy(src_hbm.at[pl.ds(off, _BB)], sbuf)

        def gb(g, _):
            o = g * 16
            sv = sbuf[pl.ds(o, 16)]
            obuf[pl.ds(o, 16)] = plsc.load_gather(dtab, [sv])
            return 0

        lax.fori_loop(0, _BB // 16, gb, 0, unroll=8)
        pltpu.sync_copy(obuf, den_out.at[pl.ds(off, _BB)])
        return 0

    lax.fori_loop(0, _NOUT, chunk_body, 0)


_SC_MESH = plsc.VectorSubcoreMesh(core_axis_name="c", subcore_axis_name="s")

_ka = functools.partial(
    pl.kernel,
    mesh=_SC_MESH,
    compiler_params=pltpu.CompilerParams(needs_layout_passes=False),
    out_type=[
        jax.ShapeDtypeStruct((_E,), jnp.float32),
        jax.ShapeDtypeStruct((_E,), jnp.float32),
        jax.ShapeDtypeStruct((_NC, _N), jnp.float32),
    ],
    scratch_types=[
        pltpu.VMEM((_N,), jnp.float32),
        pltpu.VMEM((_N,), jnp.int32),
        pltpu.VMEM((_BB,), jnp.int32),
        pltpu.VMEM((_BB,), jnp.int32),
        pltpu.VMEM((_BS, _D), jnp.float32),
        pltpu.VMEM((_BS, _D), jnp.float32),
        pltpu.VMEM((_BS, _D), jnp.float32),
        pltpu.VMEM((_BS, _D), jnp.float32),
        pltpu.VMEM((_BB,), jnp.float32),
        pltpu.VMEM((_BB,), jnp.float32),
        pltpu.VMEM((_BB,), jnp.float32),
        pltpu.VMEM_SHARED((_N,), jnp.float32),
        pltpu.SemaphoreType.DMA,
        pltpu.SemaphoreType.DMA,
        pltpu.SemaphoreType.DMA,
        pltpu.SemaphoreType.DMA,
    ],
)(_ka_body)

_kb = functools.partial(
    pl.kernel,
    mesh=_SC_MESH,
    compiler_params=pltpu.CompilerParams(needs_layout_passes=False),
    out_type=jax.ShapeDtypeStruct((_E,), jnp.float32),
    scratch_types=[
        pltpu.VMEM((_N,), jnp.float32),
        pltpu.VMEM((_N,), jnp.float32),
        pltpu.VMEM((_BB,), jnp.int32),
        pltpu.VMEM((_BB,), jnp.float32),
    ],
)(_kb_body)


def kernel(x, point_pairs, cluster_ids, recons, pts):
    src = point_pairs[0]
    dst = point_pairs[1]
    na2d = pl.pallas_call(
        _norms_body,
        out_shape=jax.ShapeDtypeStruct((_N, 1), jnp.float32),
    )(x)
    na = na2d.reshape(_N)

    expv, posf, den2 = _ka(x, src, dst, cluster_ids.astype(jnp.int32), na)
    den = _kb(den2, src)

    W = 128
    out = pl.pallas_call(
        _loss_body,
        out_shape=jax.ShapeDtypeStruct((1, 1), jnp.float32),
        out_specs=pl.BlockSpec(memory_space=pltpu.SMEM),
    )(expv.reshape(_E // W, W), den.reshape(_E // W, W),
      posf.reshape(_E // W, W))
    return out[0, 0]
